# Initial kernel scaffold; baseline (speedup 1.0000x reference)
import jax
import jax.numpy as jnp
from jax.experimental import pallas as pl


def kernel(annotation_ids, annotation_feature, annotation_edges, annotation_edges_type, W_rel, W_self, ln_gamma, ln_beta):
    raise NotImplementedError("write your pallas kernel here")


# speedup vs baseline: 11.2672x; 11.2672x over previous
import functools

import jax
import jax.numpy as jnp
from jax import lax
from jax.experimental import pallas as pl
from jax.experimental.pallas import tpu as pltpu
from jax.experimental.pallas import tpu_sc as plsc

_NC = 2
_NS = 16
_NW = _NC * _NS


def _make_sc_agg(table_rows, n_pad, d, nch, k, with_gather):
  mesh = plsc.VectorSubcoreMesh(core_axis_name="c", subcore_axis_name="s")
  rows_pt = n_pad // _NS
  lanes = 16

  scratch = [
      pltpu.VMEM((k,), jnp.int32),
      pltpu.VMEM((k, d), jnp.float32),
      pltpu.VMEM_SHARED((n_pad, d), jnp.float32),
      pltpu.SemaphoreType.DMA,
  ]
  if with_gather:
    scratch = [pltpu.VMEM((k,), jnp.int32)] + scratch

  def body(*refs):
    if with_gather:
      (table_hbm, gidx_hbm, dst_hbm, out_hbm,
       gidx_small, dst_small, rows, acc, sem) = refs
    else:
      (dst_hbm, out_hbm, dst_small, rows, acc, sem) = refs
    cid = lax.axis_index("c")
    sid = lax.axis_index("s")
    wid = cid * _NS + sid
    row0 = sid * rows_pt

    def fill(val):
      def one_row(r, _):
        for j in range(d // lanes):
          rows[r, pl.ds(j * lanes, lanes)] = jnp.full((lanes,), val,
                                                      jnp.float32)
        return 0
      lax.fori_loop(0, k, one_row, 0)

    fill(0.0)
    n_full, rem = rows_pt // k, rows_pt % k
    for t in range(n_full):
      pltpu.sync_copy(rows, acc.at[pl.ds(row0 + t * k, k)])
    if rem:
      pltpu.sync_copy(rows.at[pl.ds(0, rem)],
                      acc.at[pl.ds(row0 + n_full * k, rem)])
    plsc.subcore_barrier()

    if not with_gather:
      fill(1.0)

    e_w = nch * k

    def chunk(c, _):
      off = wid * e_w + c * k
      pltpu.sync_copy(dst_hbm.at[pl.ds(off, k)], dst_small)
      if with_gather:
        pltpu.sync_copy(gidx_hbm.at[pl.ds(off, k)], gidx_small)
        pltpu.async_copy(table_hbm.at[gidx_small], rows, sem).wait()
      pltpu.sync_copy(rows, acc.at[dst_small], add=True)
      return 0
    lax.fori_loop(0, nch, chunk, 0)

    plsc.subcore_barrier()
    pltpu.sync_copy(acc.at[pl.ds(row0, rows_pt)],
                    out_hbm.at[cid, pl.ds(row0, rows_pt)])

  return pl.kernel(
      body,
      out_type=jax.ShapeDtypeStruct((_NC, n_pad, d), jnp.float32),
      mesh=mesh,
      scratch_types=scratch,
  )


def _tc_transform(x, w_all, bn):
  rw = w_all.shape[0]
  n, d = x.shape
  nb = n // bn

  def body(x_ref, w_ref, o_ref):
    o_ref[0] = jnp.dot(x_ref[...], w_ref[0],
                       preferred_element_type=jnp.float32)

  return pl.pallas_call(
      body,
      grid=(nb, rw),
      in_specs=[
          pl.BlockSpec((bn, d), lambda b, g: (b, 0)),
          pl.BlockSpec((1, d, d), lambda b, g: (g, 0, 0)),
      ],
      out_specs=pl.BlockSpec((1, bn, d), lambda b, g: (g, b, 0)),
      out_shape=jax.ShapeDtypeStruct((rw, n, d), jnp.float32),
  )(x, w_all)


def _tc_combine(a2, s, x, deg_inv, gamma, beta, bn):
  n, d = x.shape
  nb = n // bn

  def body(a_ref, s_ref, x_ref, di_ref, g_ref, b_ref, o_ref):
    y = a_ref[0] + a_ref[1]
    h = jnp.maximum(y * di_ref[...] + s_ref[...], 0.0)
    y = h + x_ref[...]
    mu = jnp.mean(y, axis=-1, keepdims=True)
    var = jnp.mean((y - mu) * (y - mu), axis=-1, keepdims=True)
    o_ref[...] = ((y - mu) * lax.rsqrt(var + 1e-5) * g_ref[...]
                  + b_ref[...])

  return pl.pallas_call(
      body,
      grid=(nb,),
      in_specs=[
          pl.BlockSpec((2, bn, d), lambda b: (0, b, 0)),
          pl.BlockSpec((bn, d), lambda b: (b, 0)),
          pl.BlockSpec((bn, d), lambda b: (b, 0)),
          pl.BlockSpec((bn, 1), lambda b: (b, 0)),
          pl.BlockSpec((1, d), lambda b: (0, 0)),
          pl.BlockSpec((1, d), lambda b: (0, 0)),
      ],
      out_specs=pl.BlockSpec((bn, d), lambda b: (b, 0)),
      out_shape=jax.ShapeDtypeStruct((n, d), jnp.float32),
  )(a2, s, x, deg_inv, gamma, beta)


def _tc_deg_inv(a_deg2, n, bn):
  d = a_deg2.shape[2]
  nb = n // bn

  def body(a_ref, o_ref):
    deg = a_ref[0, :, :1] + a_ref[1, :, :1]
    o_ref[...] = 1.0 / jnp.maximum(deg, 1.0)

  return pl.pallas_call(
      body,
      grid=(nb,),
      in_specs=[pl.BlockSpec((2, bn, d), lambda b: (0, b, 0))],
      out_specs=pl.BlockSpec((bn, 1), lambda b: (b, 0)),
      out_shape=jax.ShapeDtypeStruct((n, 1), jnp.float32),
  )(a_deg2)


def kernel(annotation_ids, annotation_feature, annotation_edges,
           annotation_edges_type, W_rel, W_self, ln_gamma, ln_beta):
  x = annotation_feature
  n, d = x.shape
  num_layers, num_rel = W_rel.shape[0], W_rel.shape[1]
  e = annotation_edges.shape[1]

  k = 128
  e_w = pl.cdiv(e, _NW * k) * k
  e_pad = e_w * _NW
  nch = e_w // k
  n_pad = (n // 128 + 1) * 128

  src = annotation_edges[0]
  dst = annotation_edges[1]
  gidx = annotation_edges_type * n + src
  pad = e_pad - e
  gidx_flat = jnp.concatenate([gidx, jnp.zeros((pad,), jnp.int32)])
  dst_flat = jnp.concatenate([dst, jnp.full((pad,), n, jnp.int32)])

  bn = 2000
  sc_agg = _make_sc_agg(num_rel * n, n_pad, d, nch, k, with_gather=True)
  sc_deg = _make_sc_agg(0, n_pad, d, nch, k, with_gather=False)

  a_deg2 = sc_deg(dst_flat)
  deg_inv = _tc_deg_inv(a_deg2, n, bn)

  w_all = jnp.concatenate([W_rel, W_self[:, None]], axis=1)

  def step(xc, per_layer):
    w_all_l, g_l, b_l = per_layer
    hs = _tc_transform(xc, w_all_l, bn)
    h = hs[:num_rel].reshape(num_rel * n, d)
    s = hs[num_rel]
    a2 = sc_agg(h, gidx_flat, dst_flat)
    xn = _tc_combine(a2, s, xc, deg_inv, g_l[None], b_l[None], bn)
    return xn, None

  xf, _ = lax.scan(step, x, (w_all, ln_gamma, ln_beta))
  return xf
